# Initial kernel scaffold; baseline (speedup 1.0000x reference)
import jax
import jax.numpy as jnp
from jax.experimental import pallas as pl


def kernel(points, true, embedding):
    raise NotImplementedError("write your pallas kernel here")


# speedup vs baseline: 2.8178x; 2.8178x over previous
import functools

import jax
import jax.numpy as jnp
from jax import lax
from jax.experimental import pallas as pl
from jax.experimental.pallas import tpu as pltpu
from jax.experimental.pallas import tpu_sc as plsc

ALPHA = 0.7
BETA = 1.5
N = 8192
D = 64
C = 16
W = 16
PPW = N // W
NG = PPW // 16


def _sqrt16(s):
  s = jnp.maximum(s, 1e-30)
  i = lax.bitcast_convert_type(s, jnp.int32)
  i = jnp.int32(0x5F3759DF) - lax.shift_right_logical(i, 1)
  y = lax.bitcast_convert_type(i, jnp.float32)
  for _ in range(3):
    y = y * (1.5 - 0.5 * s * y * y)
  return s * y


def _saloss_body(emb_hbm, lab_hbm, pts_hbm, out_hbm,
                 emb_v, lab_v, pts_v, g_v, acc_v, sums_v, mean_v,
                 meanT_v, w_v, buf2_v, red2_v, idx80_v, idx16_v, out_v,
                 sh1_v, sh2_v):
  wid = lax.axis_index("s")
  base = wid * PPW
  iota = lax.iota(jnp.int32, 16)
  zero16 = jnp.zeros((16,), jnp.float32)

  pltpu.sync_copy(emb_hbm.at[pl.ds(base * D, PPW * D)], emb_v)
  pltpu.sync_copy(lab_hbm.at[pl.ds(base, PPW)], lab_v)
  pltpu.sync_copy(pts_hbm.at[pl.ds(base * 3, PPW * 3)], pts_v)

  def _zero_acc(r, _):
    acc_v[r, :] = zero16
    return 0
  lax.fori_loop(0, 80, _zero_acc, 0)

  def _zero_b2(r, _):
    buf2_v[r, :] = zero16
    return 0
  lax.fori_loop(0, 16, _zero_b2, 0)

  def _fill_idx(r, _):
    idx80_v[pl.ds(r * 16, 16)] = iota + r * 16
    return 0
  lax.fori_loop(0, 5, _fill_idx, 0)
  idx16_v[...] = iota

  @pl.when(wid == 0)
  def _zero_shared():
    pltpu.sync_copy(acc_v, sh1_v)
    pltpu.sync_copy(buf2_v, sh2_v)

  def _p1(j, _):
    lv = lab_v[pl.ds(j * 16, 16)]
    off = j * (16 * D)
    onehot_sum = zero16
    for jj in range(16):
      l = lv[jj]
      for k in range(4):
        v = emb_v[pl.ds(off + jj * D + k * 16, 16)]
        plsc.addupdate(acc_v.at[l * 4 + k], v)
      onehot_sum = onehot_sum + jnp.where(iota == l, 1.0, 0.0)
    plsc.addupdate(acc_v.at[64], onehot_sum)
    return 0
  lax.fori_loop(0, NG, _p1, 0)

  iota3 = iota * 3
  def _g(j, _):
    b = j * 48
    px = plsc.load_gather(pts_v, [iota3 + b])
    py = plsc.load_gather(pts_v, [iota3 + b + 1])
    pz = plsc.load_gather(pts_v, [iota3 + b + 2])
    nrm = _sqrt16(px * px + py * py + pz * pz)
    g_v[pl.ds(j * 16, 16)] = 1.0 / (1.0 + jnp.exp(-nrm))
    return 0
  lax.fori_loop(0, NG, _g, 0)

  plsc.subcore_barrier()
  pltpu.sync_copy(acc_v, sh1_v.at[idx80_v], add=True)
  plsc.subcore_barrier()
  pltpu.sync_copy(sh1_v, sums_v)

  cnt = sums_v[64, :]
  iota_f = iota.astype(jnp.float32)
  Ms = jnp.sum(jnp.where(cnt > 0.0, 1.0, 0.0))
  gate = jnp.logical_and(iota_f >= 1.0, iota_f < Ms)
  w_v[...] = jnp.where(gate, 1.0 / cnt, 0.0)

  for c in range(C):
    cntc = cnt[c]
    for k in range(4):
      r = c * 4 + k
      mean_v[pl.ds(r * 16, 16)] = sums_v[r, :] / cntc

  iota64 = iota * 64
  def _mt(dd, _):
    meanT_v[dd, :] = plsc.load_gather(mean_v, [iota64 + dd])
    return 0
  lax.fori_loop(0, D, _mt, 0)

  def _p2(j, intra):
    lv = lab_v[pl.ds(j * 16, 16)]
    lv64 = lv * 64
    eb = j * (16 * D)

    def _dim(d4, dsq):
      for k in range(4):
        d = d4 * 4 + k
        ev = plsc.load_gather(emb_v, [iota64 + (eb + d)])
        mv = plsc.load_gather(mean_v, [lv64 + d])
        diff = ev - mv
        dsq = dsq + diff * diff
      return dsq
    dsq = lax.fori_loop(0, 16, _dim, zero16)

    t = jnp.maximum(_sqrt16(dsq) - ALPHA, 0.0)
    val = g_v[pl.ds(j * 16, 16)] * t * t
    wv = plsc.load_gather(w_v, [lv])
    return intra + val * wv
  intra_vec = lax.fori_loop(0, NG, _p2, zero16)

  def _irow(d16, dsq):
    mi = mean_v[pl.ds(wid * 64 + d16 * 16, 16)]
    for k in range(16):
      diff = meanT_v[d16 * 16 + k, :] - mi[k]
      dsq = dsq + diff * diff
    return dsq
  dsq2 = lax.fori_loop(0, 4, _irow, zero16)
  hin = jnp.maximum(BETA - _sqrt16(dsq2), 0.0)
  hin = hin * hin
  gate_j = jnp.logical_and(gate, iota != wid)
  widf = lax.convert_element_type(wid, jnp.float32)
  gate_i = jnp.where(jnp.logical_and(wid >= 1, widf < Ms), 1.0, 0.0)
  inter_vec = jnp.where(gate_j, hin, 0.0) * gate_i

  buf2_v[0, :] = intra_vec
  buf2_v[1, :] = inter_vec
  pltpu.sync_copy(buf2_v, sh2_v.at[idx16_v], add=True)
  plsc.subcore_barrier()

  @pl.when(wid == 0)
  def _final():
    pltpu.sync_copy(sh2_v, red2_v)
    intra_b = jnp.full((16,), jnp.sum(red2_v[0, :]), jnp.float32)
    inter_b = jnp.full((16,), jnp.sum(red2_v[1, :]), jnp.float32)
    Mb = jnp.full((16,), Ms, jnp.float32)
    out_v[...] = intra_b / Mb + inter_b / (Mb * (Mb - 1.0))
    pltpu.sync_copy(out_v, out_hbm)


@functools.cache
def _build_saloss_sc():
  mesh = plsc.VectorSubcoreMesh(
      core_axis_name="c", subcore_axis_name="s", num_cores=1, num_subcores=16
  )
  return pl.kernel(
      _saloss_body,
      out_type=jax.ShapeDtypeStruct((16,), jnp.float32),
      mesh=mesh,
      compiler_params=pltpu.CompilerParams(
          use_tc_tiling_on_sc=False, needs_layout_passes=False
      ),
      scratch_types=[
          pltpu.VMEM((PPW * D,), jnp.float32),
          pltpu.VMEM((PPW,), jnp.int32),
          pltpu.VMEM((PPW * 3,), jnp.float32),
          pltpu.VMEM((PPW,), jnp.float32),
          pltpu.VMEM((80, 16), jnp.float32),
          pltpu.VMEM((80, 16), jnp.float32),
          pltpu.VMEM((C * D,), jnp.float32),
          pltpu.VMEM((D, 16), jnp.float32),
          pltpu.VMEM((16,), jnp.float32),
          pltpu.VMEM((16, 16), jnp.float32),
          pltpu.VMEM((16, 16), jnp.float32),
          pltpu.VMEM((80,), jnp.int32),
          pltpu.VMEM((16,), jnp.int32),
          pltpu.VMEM((16,), jnp.float32),
          pltpu.VMEM_SHARED((80, 16), jnp.float32),
          pltpu.VMEM_SHARED((16, 16), jnp.float32),
      ],
  )


def kernel(points, true, embedding):
  emb = embedding.reshape(N * D)
  lab = true.reshape(N)
  pts = points.reshape(N * 3)
  out = _build_saloss_sc()(emb, lab, pts)
  return out[:1]
